# baseline (speedup 1.0000x reference)
import jax
import jax.numpy as jnp
from jax import lax
from jax.experimental import pallas as pl
from jax.experimental.pallas import tpu as pltpu
from jax.experimental.pallas import tpu_sc as plsc

NUM_NODES = 1000000
DIM = 64
BATCH = 16384

NC = 2
NS = 16
L = 16
NW = NC * NS
B_PER_W = BATCH // NW
CH = 128
N_CHUNKS = B_PER_W // CH
N_IDX_VECS = B_PER_W // L
ROW_W = 2 * DIM


def _sc_body(src_hbm, dst_hbm, in_hbm, out_hbm, o_hbm,
             sidx_v, didx_v, srow_v, drow_v, a_t, b_t, s_v, o_v, sem):
    wid = lax.axis_index("s") * NC + lax.axis_index("c")
    base = wid * B_PER_W

    pltpu.sync_copy(src_hbm.at[pl.ds(base, B_PER_W)], sidx_v)
    pltpu.sync_copy(dst_hbm.at[pl.ds(base, B_PER_W)], didx_v)

    for i in range(N_IDX_VECS):
        sl = pl.ds(i * L, L)
        srow_v[sl] = lax.shift_right_logical(sidx_v[sl], 1)
        drow_v[sl] = lax.shift_right_logical(didx_v[sl], 1)

    iota = lax.iota(jnp.int32, L)
    col_idx = [iota * L + k for k in range(L)]

    @pl.loop(0, N_CHUNKS)
    def _chunk(g):
        lk0 = g * CH
        ca = pltpu.async_copy(in_hbm.at[srow_v.at[pl.ds(lk0, CH)]], a_t, sem)
        cb = pltpu.async_copy(out_hbm.at[drow_v.at[pl.ds(lk0, CH)]], b_t, sem)
        ca.wait()
        cb.wait()
        for grp in range(CH // L):
            svec = sidx_v[pl.ds(lk0 + grp * L, L)]
            dvec = didx_v[pl.ds(lk0 + grp * L, L)]
            soff = (svec & 1) * DIM
            doff = (dvec & 1) * DIM
            for r in range(L):
                j = grp * L + r
                ws = soff[r]
                wd = doff[r]
                p0 = a_t[j, pl.ds(ws, L)] * b_t[j, pl.ds(wd, L)]
                p1 = a_t[j, pl.ds(ws + L, L)] * b_t[j, pl.ds(wd + L, L)]
                p2 = a_t[j, pl.ds(ws + 2 * L, L)] * b_t[j, pl.ds(wd + 2 * L, L)]
                p3 = a_t[j, pl.ds(ws + 3 * L, L)] * b_t[j, pl.ds(wd + 3 * L, L)]
                s_v[pl.ds(r * L, L)] = (p0 + p1) + (p2 + p3)
            res = plsc.load_gather(s_v, [col_idx[0]])
            for k in range(1, L):
                res = res + plsc.load_gather(s_v, [col_idx[k]])
            o_v[pl.ds(lk0 + grp * L, L)] = res

    pltpu.sync_copy(o_v, o_hbm.at[pl.ds(base, B_PER_W)])


@jax.jit
def kernel(src_idx, dst_idx, in_emb, out_emb):
    mesh = plsc.VectorSubcoreMesh(
        core_axis_name="c", subcore_axis_name="s",
        num_cores=NC, num_subcores=NS)
    f = pl.kernel(
        _sc_body,
        out_type=jax.ShapeDtypeStruct((BATCH,), jnp.float32),
        mesh=mesh,
        compiler_params=pltpu.CompilerParams(needs_layout_passes=False),
        scratch_types=[
            pltpu.VMEM((B_PER_W,), jnp.int32),
            pltpu.VMEM((B_PER_W,), jnp.int32),
            pltpu.VMEM((B_PER_W,), jnp.int32),
            pltpu.VMEM((B_PER_W,), jnp.int32),
            pltpu.VMEM((CH, ROW_W), jnp.float32),
            pltpu.VMEM((CH, ROW_W), jnp.float32),
            pltpu.VMEM((L * L,), jnp.float32),
            pltpu.VMEM((B_PER_W,), jnp.float32),
            pltpu.SemaphoreType.DMA,
        ],
    )
    in2 = in_emb.reshape(NUM_NODES // 2, ROW_W)
    out2 = out_emb.reshape(NUM_NODES // 2, ROW_W)
    return f(src_idx, dst_idx, in2, out2)


# speedup vs baseline: 1.3514x; 1.3514x over previous
import jax
import jax.numpy as jnp
from jax import lax
from jax.experimental import pallas as pl
from jax.experimental.pallas import tpu as pltpu
from jax.experimental.pallas import tpu_sc as plsc

NUM_NODES = 1000000
DIM = 64
BATCH = 16384

NC = 2
NS = 16
L = 16
NW = NC * NS
B_PER_W = BATCH // NW
NBLK = (NUM_NODES + 127) // 128
BLK_PER_W = (NBLK + NW - 1) // NW
NBLK_REM = NBLK - 244 * NW
HIT_CAP = 768
N_SCAN = BATCH // L
SEN = 1 << 30


def _popcnt(m):
    r = plsc.all_reduce_population_count(m)
    return r if getattr(r, "ndim", 0) == 0 else r[0]


def _gather_body(src_hbm, dst_hbm, tin, tout, gs_out, gd_out,
                 siv, div, spk, dpk, tmp, ablk, bblk, rows,
                 sem_blk, sem_row):
    wid = lax.axis_index("s") * NC + lax.axis_index("c")
    lo = wid * 244 + jnp.minimum(wid, NBLK_REM)
    nb = 244 + (wid < NBLK_REM).astype(jnp.int32)

    pltpu.sync_copy(src_hbm, siv)
    pltpu.sync_copy(dst_hbm, div)

    iota = lax.iota(jnp.int32, L)

    for v in range(HIT_CAP // L):
        spk[pl.ds(v * L, L)] = SEN + iota * 0
        dpk[pl.ds(v * L, L)] = SEN + iota * 0

    @pl.loop(0, N_SCAN, init_carry=(0, 0))
    def _scan(g, carry):
        cs, cd = carry
        kv = g * L + iota
        sv = siv[pl.ds(g * L, L)]
        dv = div[pl.ds(g * L, L)]
        sb = lax.shift_right_logical(sv, 7)
        db = lax.shift_right_logical(dv, 7)
        ms = (sb >= lo) & (sb < lo + nb)
        md = (db >= lo) & (db < lo + nb)
        ps = ((sb - lo) << 21) | (kv << 7) | (sv & 127)
        pd = ((db - lo) << 21) | (kv << 7) | (dv & 127)
        plsc.store_compressed(spk.at[pl.ds(cs, L)], ps, mask=ms)
        plsc.store_compressed(dpk.at[pl.ds(cd, L)], pd, mask=md)
        return cs + _popcnt(ms), cd + _popcnt(md)

    ns, nd = _scan
    nsv = lax.shift_right_logical(ns + L - 1, 4)
    ndv = lax.shift_right_logical(nd + L - 1, 4)

    def _emit_hits(pk, nv, blk_buf, g_out, blo, bhi, issued0):
        def scan_vreg(v, issued):
            pv = pk[pl.ds(v * L, L)]
            m = (pv >= blo) & (pv < bhi)
            cnt = _popcnt(m)
            plsc.store_compressed(tmp.at[pl.ds(0, L)], pv, mask=m)

            def hit(j, iss):
                pj = plsc.load_gather(tmp, [iota * 0 + j])[0]
                k = lax.shift_right_logical(pj, 7) & jnp.int32(16383)
                lane = pj & jnp.int32(127)
                slot = iss & 7

                @pl.when(iss >= 8)
                def _():
                    pltpu.make_async_copy(
                        rows.at[0], g_out.at[pl.ds(0, DIM)], sem_row).wait()

                for c in range(4):
                    rows[slot, pl.ds(c * L, L)] = plsc.load_gather(
                        blk_buf, [iota + c * L, iota * 0 + lane])
                koff = pl.multiple_of(k * DIM, DIM)
                pltpu.async_copy(
                    rows.at[slot], g_out.at[pl.ds(koff, DIM)], sem_row)
                return iss + 1

            return lax.fori_loop(0, cnt, hit, issued)

        return lax.fori_loop(0, nv, scan_vreg, issued0)

    @pl.loop(0, BLK_PER_W, init_carry=0)
    def _blocks(b, issued):
        bb = jnp.minimum(b, nb - 1)
        off = pl.multiple_of((lo + bb) * 128, 128)
        ca = pltpu.async_copy(
            tin.at[pl.ds(0, DIM), pl.ds(off, 128)], ablk, sem_blk)
        cb = pltpu.async_copy(
            tout.at[pl.ds(0, DIM), pl.ds(off, 128)], bblk, sem_blk)
        ca.wait()
        cb.wait()
        blo = bb << 21
        bhi = (bb + 1) << 21
        issued = _emit_hits(spk, nsv, ablk, gs_out, blo, bhi, issued)
        issued = _emit_hits(dpk, ndv, bblk, gd_out, blo, bhi, issued)
        return issued

    issued = _blocks

    def drain(j, x):
        pltpu.make_async_copy(
            rows.at[0], gs_out.at[pl.ds(0, DIM)], sem_row).wait()
        return x

    lax.fori_loop(0, jnp.minimum(issued, 8), drain, 0)


def _dot_body(gs, gd, o_hbm, a_v, b_v, s_v, o_v):
    wid = lax.axis_index("s") * NC + lax.axis_index("c")
    base = wid * B_PER_W

    pltpu.sync_copy(gs.at[pl.ds(base * DIM, B_PER_W * DIM)], a_v)
    pltpu.sync_copy(gd.at[pl.ds(base * DIM, B_PER_W * DIM)], b_v)

    iota = lax.iota(jnp.int32, L)
    col_idx = [iota * L + k for k in range(L)]

    @pl.loop(0, B_PER_W // L)
    def _group(g):
        row0 = g * L
        for r in range(L):
            w = (row0 + r) * DIM
            p0 = a_v[pl.ds(w, L)] * b_v[pl.ds(w, L)]
            p1 = a_v[pl.ds(w + L, L)] * b_v[pl.ds(w + L, L)]
            p2 = a_v[pl.ds(w + 2 * L, L)] * b_v[pl.ds(w + 2 * L, L)]
            p3 = a_v[pl.ds(w + 3 * L, L)] * b_v[pl.ds(w + 3 * L, L)]
            s_v[pl.ds(r * L, L)] = (p0 + p1) + (p2 + p3)
        res = plsc.load_gather(s_v, [col_idx[0]])
        for k in range(1, L):
            res = res + plsc.load_gather(s_v, [col_idx[k]])
        o_v[pl.ds(row0, L)] = res

    pltpu.sync_copy(o_v, o_hbm.at[pl.ds(base, B_PER_W)])


@jax.jit
def kernel(src_idx, dst_idx, in_emb, out_emb):
    mesh = plsc.VectorSubcoreMesh(
        core_axis_name="c", subcore_axis_name="s",
        num_cores=NC, num_subcores=NS)
    cp = pltpu.CompilerParams(needs_layout_passes=False)

    k1 = pl.kernel(
        _gather_body,
        out_type=(
            jax.ShapeDtypeStruct((BATCH * DIM,), jnp.float32),
            jax.ShapeDtypeStruct((BATCH * DIM,), jnp.float32),
        ),
        mesh=mesh,
        compiler_params=cp,
        scratch_types=[
            pltpu.VMEM((BATCH,), jnp.int32),
            pltpu.VMEM((BATCH,), jnp.int32),
            pltpu.VMEM((HIT_CAP,), jnp.int32),
            pltpu.VMEM((HIT_CAP,), jnp.int32),
            pltpu.VMEM((L,), jnp.int32),
            pltpu.VMEM((DIM, 128), jnp.float32),
            pltpu.VMEM((DIM, 128), jnp.float32),
            pltpu.VMEM((8, DIM), jnp.float32),
            pltpu.SemaphoreType.DMA,
            pltpu.SemaphoreType.DMA,
        ],
    )
    k2 = pl.kernel(
        _dot_body,
        out_type=jax.ShapeDtypeStruct((BATCH,), jnp.float32),
        mesh=mesh,
        compiler_params=cp,
        scratch_types=[
            pltpu.VMEM((B_PER_W * DIM,), jnp.float32),
            pltpu.VMEM((B_PER_W * DIM,), jnp.float32),
            pltpu.VMEM((L * L,), jnp.float32),
            pltpu.VMEM((B_PER_W,), jnp.float32),
        ],
    )
    gs, gd = k1(src_idx, dst_idx, in_emb.T, out_emb.T)
    return k2(gs, gd)


# speedup vs baseline: 3.6614x; 2.7093x over previous
import jax
import jax.numpy as jnp
from jax import lax
from jax.experimental import pallas as pl
from jax.experimental.pallas import tpu as pltpu
from jax.experimental.pallas import tpu_sc as plsc

NUM_NODES = 1000000
DIM = 64
BATCH = 16384

NC = 2
NS = 16
L = 16
NW = NC * NS
B_PER_W = BATCH // NW
NBLK = (NUM_NODES + 127) // 128
NBLK_REM = NBLK - 244 * NW
GRP = 8
N_GRP = 31
HIT_CAP = 768
GCAP = 64
N_SCAN = BATCH // (2 * L)
SEN = 1 << 30


def _popcnt(m):
    r = plsc.all_reduce_population_count(m)
    return r if getattr(r, "ndim", 0) == 0 else r[0]


def _gather_body(src_hbm, dst_hbm, tin, tout, gs_out, gd_out,
                 siv, div, spk, dpk, gts, gtd, tmp,
                 ablk0, bblk0, ablk1, bblk1, rows,
                 sem_blk, sem_row):
    wid = lax.axis_index("s") * NC + lax.axis_index("c")
    lo = wid * 244 + jnp.minimum(wid, NBLK_REM)
    nb = 244 + (wid < NBLK_REM).astype(jnp.int32)

    pltpu.sync_copy(src_hbm, siv)
    pltpu.sync_copy(dst_hbm, div)

    iota = lax.iota(jnp.int32, L)

    for v in range(HIT_CAP // L):
        spk[pl.ds(v * L, L)] = SEN + iota * 0
        dpk[pl.ds(v * L, L)] = SEN + iota * 0

    @pl.loop(0, N_SCAN, init_carry=(0, 0))
    def _scan(g, carry):
        cs, cd = carry
        for h in range(2):
            base = (2 * g + h) * L
            kv = base + iota
            sv = siv[pl.ds(base, L)]
            dv = div[pl.ds(base, L)]
            sb = lax.shift_right_logical(sv, 7)
            db = lax.shift_right_logical(dv, 7)
            ms = (sb >= lo) & (sb < lo + nb)
            md = (db >= lo) & (db < lo + nb)
            ps = ((sb - lo) << 21) | (kv << 7) | (sv & 127)
            pd = ((db - lo) << 21) | (kv << 7) | (dv & 127)
            plsc.store_compressed(spk.at[pl.ds(cs, L)], ps, mask=ms)
            plsc.store_compressed(dpk.at[pl.ds(cd, L)], pd, mask=md)
            cs = cs + _popcnt(ms)
            cd = cd + _popcnt(md)
        return cs, cd

    ns, nd = _scan
    nsv = lax.shift_right_logical(ns + L - 1, 4)
    ndv = lax.shift_right_logical(nd + L - 1, 4)

    abufs = [ablk0, ablk1]
    bbufs = [bblk0, bblk1]

    def fetch(blk, a_buf, b_buf):
        off = pl.multiple_of(jnp.minimum(lo + blk, lo + nb - 1) * 128, 128)
        pltpu.async_copy(tin.at[pl.ds(0, DIM), pl.ds(off, 128)], a_buf, sem_blk)
        pltpu.async_copy(tout.at[pl.ds(0, DIM), pl.ds(off, 128)], b_buf, sem_blk)

    def wait_pair(a_buf, b_buf):
        pltpu.make_async_copy(
            tin.at[pl.ds(0, DIM), pl.ds(0, 128)], a_buf, sem_blk).wait()
        pltpu.make_async_copy(
            tout.at[pl.ds(0, DIM), pl.ds(0, 128)], b_buf, sem_blk).wait()

    def collect(pk, nv, dst_ref, blo, bhi):
        def body(v, cur):
            pv = pk[pl.ds(v * L, L)]
            m = (pv >= blo) & (pv < bhi)
            plsc.store_compressed(dst_ref.at[pl.ds(cur, L)], pv, mask=m)
            return cur + _popcnt(m)
        return lax.fori_loop(0, nv, body, jnp.int32(0))

    def emit(gt_ref, cg, blk_buf, g_out, blo, bhi, issued0):
        nv = lax.shift_right_logical(cg + L - 1, 4)

        def scan_vreg(v, issued):
            pv = gt_ref[pl.ds(v * L, L)]
            m = (pv >= blo) & (pv < bhi)
            cnt = _popcnt(m)
            plsc.store_compressed(tmp.at[pl.ds(0, L)], pv, mask=m)

            def hit(j, iss):
                pj = plsc.load_gather(tmp, [iota * 0 + j])[0]
                k = lax.shift_right_logical(pj, 7) & jnp.int32(16383)
                lane = pj & jnp.int32(127)
                slot = iss & 7

                @pl.when(iss >= 8)
                def _():
                    pltpu.make_async_copy(
                        rows.at[0], g_out.at[pl.ds(0, DIM)], sem_row).wait()

                for c in range(4):
                    rows[slot, pl.ds(c * L, L)] = plsc.load_gather(
                        blk_buf, [iota + c * L, iota * 0 + lane])
                koff = pl.multiple_of(k * DIM, DIM)
                pltpu.async_copy(
                    rows.at[slot], g_out.at[pl.ds(koff, DIM)], sem_row)
                return iss + 1

            return lax.fori_loop(0, cnt, hit, issued)

        return lax.fori_loop(0, nv, scan_vreg, issued0)

    fetch(0, abufs[0], bbufs[0])

    @pl.loop(0, N_GRP, init_carry=0)
    def _groups(g, issued):
        g0 = g * GRP
        glo = g0 << 21
        ghi = (g0 + GRP) << 21
        cgs = collect(spk, nsv, gts, glo, ghi)
        cgd = collect(dpk, ndv, gtd, glo, ghi)
        for bi in range(GRP):
            b = g0 + bi
            cur = bi & 1
            nxt = cur ^ 1
            fetch(b + 1, abufs[nxt], bbufs[nxt])
            wait_pair(abufs[cur], bbufs[cur])
            bb = jnp.minimum(b, nb - 1)
            blo = bb << 21
            bhi = (bb + 1) << 21
            issued = emit(gts, cgs, abufs[cur], gs_out, blo, bhi, issued)
            issued = emit(gtd, cgd, bbufs[cur], gd_out, blo, bhi, issued)
        return issued

    issued = _groups

    wait_pair(abufs[0], bbufs[0])

    def drain(j, x):
        pltpu.make_async_copy(
            rows.at[0], gs_out.at[pl.ds(0, DIM)], sem_row).wait()
        return x

    lax.fori_loop(0, jnp.minimum(issued, 8), drain, 0)


def _dot_body(gs, gd, o_hbm, a_v, b_v, s_v, o_v):
    wid = lax.axis_index("s") * NC + lax.axis_index("c")
    base = wid * B_PER_W

    pltpu.sync_copy(gs.at[pl.ds(base * DIM, B_PER_W * DIM)], a_v)
    pltpu.sync_copy(gd.at[pl.ds(base * DIM, B_PER_W * DIM)], b_v)

    iota = lax.iota(jnp.int32, L)
    col_idx = [iota * L + k for k in range(L)]

    @pl.loop(0, B_PER_W // L)
    def _group(g):
        row0 = g * L
        for r in range(L):
            w = (row0 + r) * DIM
            p0 = a_v[pl.ds(w, L)] * b_v[pl.ds(w, L)]
            p1 = a_v[pl.ds(w + L, L)] * b_v[pl.ds(w + L, L)]
            p2 = a_v[pl.ds(w + 2 * L, L)] * b_v[pl.ds(w + 2 * L, L)]
            p3 = a_v[pl.ds(w + 3 * L, L)] * b_v[pl.ds(w + 3 * L, L)]
            s_v[pl.ds(r * L, L)] = (p0 + p1) + (p2 + p3)
        res = plsc.load_gather(s_v, [col_idx[0]])
        for k in range(1, L):
            res = res + plsc.load_gather(s_v, [col_idx[k]])
        o_v[pl.ds(row0, L)] = res

    pltpu.sync_copy(o_v, o_hbm.at[pl.ds(base, B_PER_W)])


@jax.jit
def kernel(src_idx, dst_idx, in_emb, out_emb):
    mesh = plsc.VectorSubcoreMesh(
        core_axis_name="c", subcore_axis_name="s",
        num_cores=NC, num_subcores=NS)
    cp = pltpu.CompilerParams(needs_layout_passes=False)

    k1 = pl.kernel(
        _gather_body,
        out_type=(
            jax.ShapeDtypeStruct((BATCH * DIM,), jnp.float32),
            jax.ShapeDtypeStruct((BATCH * DIM,), jnp.float32),
        ),
        mesh=mesh,
        compiler_params=cp,
        scratch_types=[
            pltpu.VMEM((BATCH,), jnp.int32),
            pltpu.VMEM((BATCH,), jnp.int32),
            pltpu.VMEM((HIT_CAP,), jnp.int32),
            pltpu.VMEM((HIT_CAP,), jnp.int32),
            pltpu.VMEM((GCAP,), jnp.int32),
            pltpu.VMEM((GCAP,), jnp.int32),
            pltpu.VMEM((L,), jnp.int32),
            pltpu.VMEM((DIM, 128), jnp.float32),
            pltpu.VMEM((DIM, 128), jnp.float32),
            pltpu.VMEM((DIM, 128), jnp.float32),
            pltpu.VMEM((DIM, 128), jnp.float32),
            pltpu.VMEM((8, DIM), jnp.float32),
            pltpu.SemaphoreType.DMA,
            pltpu.SemaphoreType.DMA,
        ],
    )
    k2 = pl.kernel(
        _dot_body,
        out_type=jax.ShapeDtypeStruct((BATCH,), jnp.float32),
        mesh=mesh,
        compiler_params=cp,
        scratch_types=[
            pltpu.VMEM((B_PER_W * DIM,), jnp.float32),
            pltpu.VMEM((B_PER_W * DIM,), jnp.float32),
            pltpu.VMEM((L * L,), jnp.float32),
            pltpu.VMEM((B_PER_W,), jnp.float32),
        ],
    )
    gs, gd = k1(src_idx, dst_idx, in_emb.T, out_emb.T)
    return k2(gs, gd)


# speedup vs baseline: 3.7132x; 1.0141x over previous
import jax
import jax.numpy as jnp
from jax import lax
from jax.experimental import pallas as pl
from jax.experimental.pallas import tpu as pltpu
from jax.experimental.pallas import tpu_sc as plsc

NUM_NODES = 1000000
DIM = 64
BATCH = 16384

NC = 2
NS = 16
L = 16
NW = NC * NS
B_PER_W = BATCH // NW
NBLK = (NUM_NODES + 127) // 128
NBLK_REM = NBLK - 244 * NW
GRP = 8
N_GRP2 = 16
HIT_CAP = 768
GCAP = 64
N_SCAN = BATCH // (2 * L)
SEN = 1 << 30


def _popcnt(m):
    r = plsc.all_reduce_population_count(m)
    return r if getattr(r, "ndim", 0) == 0 else r[0]


def _gather_body(src_hbm, dst_hbm, tin, tout, gs_out, gd_out,
                 siv, div, spk, dpk, gts0, gtd0, gts1, gtd1, tmp,
                 ablk0, bblk0, ablk1, bblk1, rows,
                 sem_blk, sem_row):
    wid = lax.axis_index("s") * NC + lax.axis_index("c")
    lo = wid * 244 + jnp.minimum(wid, NBLK_REM)
    nb = 244 + (wid < NBLK_REM).astype(jnp.int32)

    pltpu.sync_copy(src_hbm, siv)
    pltpu.sync_copy(dst_hbm, div)

    iota = lax.iota(jnp.int32, L)

    for v in range(HIT_CAP // L):
        spk[pl.ds(v * L, L)] = SEN + iota * 0
        dpk[pl.ds(v * L, L)] = SEN + iota * 0

    @pl.loop(0, N_SCAN, init_carry=(0, 0))
    def _scan(g, carry):
        cs, cd = carry
        for h in range(2):
            base = (2 * g + h) * L
            kv = base + iota
            sv = siv[pl.ds(base, L)]
            dv = div[pl.ds(base, L)]
            sb = lax.shift_right_logical(sv, 7)
            db = lax.shift_right_logical(dv, 7)
            ms = (sb >= lo) & (sb < lo + nb)
            md = (db >= lo) & (db < lo + nb)
            ps = ((sb - lo) << 21) | (kv << 7) | (sv & 127)
            pd = ((db - lo) << 21) | (kv << 7) | (dv & 127)
            plsc.store_compressed(spk.at[pl.ds(cs, L)], ps, mask=ms)
            plsc.store_compressed(dpk.at[pl.ds(cd, L)], pd, mask=md)
            cs = cs + _popcnt(ms)
            cd = cd + _popcnt(md)
        return cs, cd

    ns, nd = _scan
    nsv = lax.shift_right_logical(ns + L - 1, 4)
    ndv = lax.shift_right_logical(nd + L - 1, 4)

    abufs = [ablk0, ablk1]
    bbufs = [bblk0, bblk1]
    gts = [gts0, gts1]
    gtd = [gtd0, gtd1]

    def fetch1(tbl, blk, buf, pred):
        @pl.when(pred)
        def _():
            off = pl.multiple_of(
                jnp.minimum(lo + blk, NBLK - 1) * 128, 128)
            pltpu.async_copy(
                tbl.at[pl.ds(0, DIM), pl.ds(off, 128)], buf, sem_blk)

    def wait1(tbl, buf, pred):
        @pl.when(pred)
        def _():
            pltpu.make_async_copy(
                tbl.at[pl.ds(0, DIM), pl.ds(0, 128)], buf, sem_blk).wait()

    def collect(pk, nv, dst_ref, g0):
        blo = g0 << 21
        bhi = (g0 + GRP) << 21

        def body(v, carry):
            cur, bv = carry
            pv = pk[pl.ds(v * L, L)]
            m = (pv >= blo) & (pv < bhi)
            plsc.store_compressed(dst_ref.at[pl.ds(cur, L)], pv, mask=m)
            bit = jnp.where(
                m, 1 << ((lax.shift_right_logical(pv, 21) - g0) & 7), 0)
            return cur + _popcnt(m), bv | bit

        iota0 = lax.iota(jnp.int32, L) * 0
        cur, bv = lax.fori_loop(0, nv, body, (jnp.int32(0), iota0))
        msk = jnp.int32(0)
        for bi in range(GRP):
            has = (_popcnt((bv & (1 << bi)) != 0) > 0).astype(jnp.int32)
            msk = msk | (has << bi)
        return cur, msk

    def emit(gt_ref, cg, blk_buf, g_out, blo, bhi, issued0):
        nv = lax.shift_right_logical(cg + L - 1, 4)

        def scan_vreg(v, issued):
            pv = gt_ref[pl.ds(v * L, L)]
            m = (pv >= blo) & (pv < bhi)
            cnt = _popcnt(m)
            plsc.store_compressed(tmp.at[pl.ds(0, L)], pv, mask=m)

            def hit(j, iss):
                pj = plsc.load_gather(tmp, [iota * 0 + j])[0]
                k = lax.shift_right_logical(pj, 7) & jnp.int32(16383)
                lane = pj & jnp.int32(127)
                slot = iss & 7

                @pl.when(iss >= 8)
                def _():
                    pltpu.make_async_copy(
                        rows.at[0], g_out.at[pl.ds(0, DIM)], sem_row).wait()

                for c in range(4):
                    rows[slot, pl.ds(c * L, L)] = plsc.load_gather(
                        blk_buf, [iota + c * L, iota * 0 + lane])
                koff = pl.multiple_of(k * DIM, DIM)
                pltpu.async_copy(
                    rows.at[slot], g_out.at[pl.ds(koff, DIM)], sem_row)
                return iss + 1

            return lax.fori_loop(0, cnt, hit, issued)

        return lax.fori_loop(0, nv, scan_vreg, issued0)

    cgs0, ms0 = collect(spk, nsv, gts[0], jnp.int32(0))
    cgd0, md0 = collect(dpk, ndv, gtd[0], jnp.int32(0))
    fetch1(tin, jnp.int32(0), abufs[0], (ms0 & 1) > 0)
    fetch1(tout, jnp.int32(0), bbufs[0], (md0 & 1) > 0)

    @pl.loop(0, N_GRP2, init_carry=(0, cgs0, cgd0, ms0, md0))
    def _groups(t, carry):
        issued, cgs, cgd, msks, mskd = carry
        for p in range(2):
            g = 2 * t + p
            g0 = g * GRP
            cgs_nx = cgd_nx = msks_nx = mskd_nx = None
            for bi in range(GRP):
                b = g0 + bi
                cur = bi & 1
                nxt = cur ^ 1
                if bi < GRP - 1:
                    fs_n = ((msks >> (bi + 1)) & 1) > 0
                    fd_n = ((mskd >> (bi + 1)) & 1) > 0
                else:
                    fs_n = (msks_nx & 1) > 0
                    fd_n = (mskd_nx & 1) > 0
                fetch1(tin, b + 1, abufs[nxt], fs_n)
                fetch1(tout, b + 1, bbufs[nxt], fd_n)
                wait1(tin, abufs[cur], ((msks >> bi) & 1) > 0)
                wait1(tout, bbufs[cur], ((mskd >> bi) & 1) > 0)
                blo = b << 21
                bhi = (b + 1) << 21
                issued = emit(
                    gts[p], cgs, abufs[cur], gs_out, blo, bhi, issued)
                issued = emit(
                    gtd[p], cgd, bbufs[cur], gd_out, blo, bhi, issued)
                if bi == 0:
                    gnx = (g + 1) * GRP
                    cgs_nx, msks_nx = collect(spk, nsv, gts[p ^ 1], gnx)
                    cgd_nx, mskd_nx = collect(dpk, ndv, gtd[p ^ 1], gnx)
            cgs, cgd, msks, mskd = cgs_nx, cgd_nx, msks_nx, mskd_nx
        return issued, cgs, cgd, msks, mskd

    issued = _groups[0]

    def drain(j, x):
        pltpu.make_async_copy(
            rows.at[0], gs_out.at[pl.ds(0, DIM)], sem_row).wait()
        return x

    lax.fori_loop(0, jnp.minimum(issued, 8), drain, 0)


def _dot_body(gs, gd, o_hbm, a_v, b_v, s_v, o_v):
    wid = lax.axis_index("s") * NC + lax.axis_index("c")
    base = wid * B_PER_W

    pltpu.sync_copy(gs.at[pl.ds(base * DIM, B_PER_W * DIM)], a_v)
    pltpu.sync_copy(gd.at[pl.ds(base * DIM, B_PER_W * DIM)], b_v)

    iota = lax.iota(jnp.int32, L)
    col_idx = [iota * L + k for k in range(L)]

    @pl.loop(0, B_PER_W // L)
    def _group(g):
        row0 = g * L
        for r in range(L):
            w = (row0 + r) * DIM
            p0 = a_v[pl.ds(w, L)] * b_v[pl.ds(w, L)]
            p1 = a_v[pl.ds(w + L, L)] * b_v[pl.ds(w + L, L)]
            p2 = a_v[pl.ds(w + 2 * L, L)] * b_v[pl.ds(w + 2 * L, L)]
            p3 = a_v[pl.ds(w + 3 * L, L)] * b_v[pl.ds(w + 3 * L, L)]
            s_v[pl.ds(r * L, L)] = (p0 + p1) + (p2 + p3)
        res = plsc.load_gather(s_v, [col_idx[0]])
        for k in range(1, L):
            res = res + plsc.load_gather(s_v, [col_idx[k]])
        o_v[pl.ds(row0, L)] = res

    pltpu.sync_copy(o_v, o_hbm.at[pl.ds(base, B_PER_W)])


@jax.jit
def kernel(src_idx, dst_idx, in_emb, out_emb):
    mesh = plsc.VectorSubcoreMesh(
        core_axis_name="c", subcore_axis_name="s",
        num_cores=NC, num_subcores=NS)
    cp = pltpu.CompilerParams(needs_layout_passes=False)

    k1 = pl.kernel(
        _gather_body,
        out_type=(
            jax.ShapeDtypeStruct((BATCH * DIM,), jnp.float32),
            jax.ShapeDtypeStruct((BATCH * DIM,), jnp.float32),
        ),
        mesh=mesh,
        compiler_params=cp,
        scratch_types=[
            pltpu.VMEM((BATCH,), jnp.int32),
            pltpu.VMEM((BATCH,), jnp.int32),
            pltpu.VMEM((HIT_CAP,), jnp.int32),
            pltpu.VMEM((HIT_CAP,), jnp.int32),
            pltpu.VMEM((GCAP,), jnp.int32),
            pltpu.VMEM((GCAP,), jnp.int32),
            pltpu.VMEM((GCAP,), jnp.int32),
            pltpu.VMEM((GCAP,), jnp.int32),
            pltpu.VMEM((L,), jnp.int32),
            pltpu.VMEM((DIM, 128), jnp.float32),
            pltpu.VMEM((DIM, 128), jnp.float32),
            pltpu.VMEM((DIM, 128), jnp.float32),
            pltpu.VMEM((DIM, 128), jnp.float32),
            pltpu.VMEM((8, DIM), jnp.float32),
            pltpu.SemaphoreType.DMA,
            pltpu.SemaphoreType.DMA,
        ],
    )
    k2 = pl.kernel(
        _dot_body,
        out_type=jax.ShapeDtypeStruct((BATCH,), jnp.float32),
        mesh=mesh,
        compiler_params=cp,
        scratch_types=[
            pltpu.VMEM((B_PER_W * DIM,), jnp.float32),
            pltpu.VMEM((B_PER_W * DIM,), jnp.float32),
            pltpu.VMEM((L * L,), jnp.float32),
            pltpu.VMEM((B_PER_W,), jnp.float32),
        ],
    )
    gs, gd = k1(src_idx, dst_idx, in_emb.T, out_emb.T)
    return k2(gs, gd)


# speedup vs baseline: 4.2745x; 1.1512x over previous
import jax
import jax.numpy as jnp
from jax import lax
from jax.experimental import pallas as pl
from jax.experimental.pallas import tpu as pltpu
from jax.experimental.pallas import tpu_sc as plsc

NUM_NODES = 1000000
DIM = 64
BATCH = 16384

NC = 2
NS = 16
L = 16
NW = NC * NS
B_PER_W = BATCH // NW
NBLK = (NUM_NODES + 127) // 128
NBLK_REM = NBLK - 244 * NW
GRP = 8
N_GRP2 = 16
HIT_CAP = 768
GCAP = 64
N_SCAN = BATCH // (2 * L)
SEN = 1 << 30


def _popcnt(m):
    r = plsc.all_reduce_population_count(m)
    return r if getattr(r, "ndim", 0) == 0 else r[0]


def _gather_body(src_hbm, dst_hbm, tin, tout, gs_out, gd_out,
                 siv, div, spk, dpk, gts0, gtd0, gts1, gtd1, tmp,
                 ablk0, bblk0, ablk1, bblk1, ablk2, bblk2, ablk3, bblk3, rows,
                 sem_blk, sem_row):
    wid = lax.axis_index("s") * NC + lax.axis_index("c")
    lo = wid * 244 + jnp.minimum(wid, NBLK_REM)
    nb = 244 + (wid < NBLK_REM).astype(jnp.int32)

    pltpu.sync_copy(src_hbm, siv)
    pltpu.sync_copy(dst_hbm, div)

    iota = lax.iota(jnp.int32, L)

    for v in range(HIT_CAP // L):
        spk[pl.ds(v * L, L)] = SEN + iota * 0
        dpk[pl.ds(v * L, L)] = SEN + iota * 0

    @pl.loop(0, N_SCAN, init_carry=(0, 0))
    def _scan(g, carry):
        cs, cd = carry
        for h in range(2):
            base = (2 * g + h) * L
            kv = base + iota
            sv = siv[pl.ds(base, L)]
            dv = div[pl.ds(base, L)]
            sb = lax.shift_right_logical(sv, 7)
            db = lax.shift_right_logical(dv, 7)
            ms = (sb >= lo) & (sb < lo + nb)
            md = (db >= lo) & (db < lo + nb)
            ps = ((sb - lo) << 21) | (kv << 7) | (sv & 127)
            pd = ((db - lo) << 21) | (kv << 7) | (dv & 127)
            plsc.store_compressed(spk.at[pl.ds(cs, L)], ps, mask=ms)
            plsc.store_compressed(dpk.at[pl.ds(cd, L)], pd, mask=md)
            cs = cs + _popcnt(ms)
            cd = cd + _popcnt(md)
        return cs, cd

    ns, nd = _scan
    nsv = lax.shift_right_logical(ns + L - 1, 4)
    ndv = lax.shift_right_logical(nd + L - 1, 4)

    abufs = [ablk0, ablk1, ablk2, ablk3]
    bbufs = [bblk0, bblk1, bblk2, bblk3]
    gts = [gts0, gts1]
    gtd = [gtd0, gtd1]

    def fetch1(tbl, blk, buf, pred):
        @pl.when(pred)
        def _():
            off = pl.multiple_of(
                jnp.minimum(lo + blk, NBLK - 1) * 128, 128)
            pltpu.async_copy(
                tbl.at[pl.ds(0, DIM), pl.ds(off, 128)], buf, sem_blk)

    def wait1(tbl, buf, pred):
        @pl.when(pred)
        def _():
            pltpu.make_async_copy(
                tbl.at[pl.ds(0, DIM), pl.ds(0, 128)], buf, sem_blk).wait()

    def collect(pk, nv, dst_ref, g0):
        blo = g0 << 21
        bhi = (g0 + GRP) << 21

        def body(v, carry):
            cur, bv = carry
            pv = pk[pl.ds(v * L, L)]
            m = (pv >= blo) & (pv < bhi)
            plsc.store_compressed(dst_ref.at[pl.ds(cur, L)], pv, mask=m)
            bit = jnp.where(
                m, 1 << ((lax.shift_right_logical(pv, 21) - g0) & 7), 0)
            return cur + _popcnt(m), bv | bit

        iota0 = lax.iota(jnp.int32, L) * 0
        cur, bv = lax.fori_loop(0, nv, body, (jnp.int32(0), iota0))
        msk = jnp.int32(0)
        for bi in range(GRP):
            has = (_popcnt((bv & (1 << bi)) != 0) > 0).astype(jnp.int32)
            msk = msk | (has << bi)
        return cur, msk

    def emit(gt_ref, cg, blk_buf, g_out, blo, bhi, issued0):
        nv = lax.shift_right_logical(cg + L - 1, 4)

        def scan_vreg(v, issued):
            pv = gt_ref[pl.ds(v * L, L)]
            m = (pv >= blo) & (pv < bhi)
            cnt = _popcnt(m)
            plsc.store_compressed(tmp.at[pl.ds(0, L)], pv, mask=m)

            def hit(j, iss):
                pj = plsc.load_gather(tmp, [iota * 0 + j])[0]
                k = lax.shift_right_logical(pj, 7) & jnp.int32(16383)
                lane = pj & jnp.int32(127)
                slot = iss & 7

                @pl.when(iss >= 8)
                def _():
                    pltpu.make_async_copy(
                        rows.at[0], g_out.at[pl.ds(0, DIM)], sem_row).wait()

                for c in range(4):
                    rows[slot, pl.ds(c * L, L)] = plsc.load_gather(
                        blk_buf, [iota + c * L, iota * 0 + lane])
                koff = pl.multiple_of(k * DIM, DIM)
                pltpu.async_copy(
                    rows.at[slot], g_out.at[pl.ds(koff, DIM)], sem_row)
                return iss + 1

            return lax.fori_loop(0, cnt, hit, issued)

        return lax.fori_loop(0, nv, scan_vreg, issued0)

    cgs0, ms0 = collect(spk, nsv, gts[0], jnp.int32(0))
    cgd0, md0 = collect(dpk, ndv, gtd[0], jnp.int32(0))
    fetch1(tin, jnp.int32(0), abufs[0], (ms0 & 1) > 0)
    fetch1(tout, jnp.int32(0), bbufs[0], (md0 & 1) > 0)
    fetch1(tin, jnp.int32(1), abufs[1], (ms0 & 2) > 0)
    fetch1(tout, jnp.int32(1), bbufs[1], (md0 & 2) > 0)

    @pl.loop(0, N_GRP2, init_carry=(0, cgs0, cgd0, ms0, md0))
    def _groups(t, carry):
        issued, cgs, cgd, msks, mskd = carry
        for p in range(2):
            g = 2 * t + p
            g0 = g * GRP
            cgs_nx = cgd_nx = msks_nx = mskd_nx = None
            for bi in range(GRP):
                b = g0 + bi
                cur = bi % 4
                nxt = (bi + 2) % 4
                if bi < GRP - 2:
                    fs_n = ((msks >> (bi + 2)) & 1) > 0
                    fd_n = ((mskd >> (bi + 2)) & 1) > 0
                else:
                    fs_n = ((msks_nx >> (bi - 6)) & 1) > 0
                    fd_n = ((mskd_nx >> (bi - 6)) & 1) > 0
                fetch1(tin, b + 2, abufs[nxt], fs_n)
                fetch1(tout, b + 2, bbufs[nxt], fd_n)
                wait1(tin, abufs[cur], ((msks >> bi) & 1) > 0)
                wait1(tout, bbufs[cur], ((mskd >> bi) & 1) > 0)
                blo = b << 21
                bhi = (b + 1) << 21
                issued = emit(
                    gts[p], cgs, abufs[cur], gs_out, blo, bhi, issued)
                issued = emit(
                    gtd[p], cgd, bbufs[cur], gd_out, blo, bhi, issued)
                if bi == 0:
                    gnx = (g + 1) * GRP
                    cgs_nx, msks_nx = collect(spk, nsv, gts[p ^ 1], gnx)
                    cgd_nx, mskd_nx = collect(dpk, ndv, gtd[p ^ 1], gnx)
            cgs, cgd, msks, mskd = cgs_nx, cgd_nx, msks_nx, mskd_nx
        return issued, cgs, cgd, msks, mskd

    issued = _groups[0]

    def drain(j, x):
        pltpu.make_async_copy(
            rows.at[0], gs_out.at[pl.ds(0, DIM)], sem_row).wait()
        return x

    lax.fori_loop(0, jnp.minimum(issued, 8), drain, 0)


def _dot_body(gs, gd, o_hbm, a_v, b_v, s_v, o_v):
    wid = lax.axis_index("s") * NC + lax.axis_index("c")
    base = wid * B_PER_W

    pltpu.sync_copy(gs.at[pl.ds(base * DIM, B_PER_W * DIM)], a_v)
    pltpu.sync_copy(gd.at[pl.ds(base * DIM, B_PER_W * DIM)], b_v)

    iota = lax.iota(jnp.int32, L)
    col_idx = [iota * L + k for k in range(L)]

    @pl.loop(0, B_PER_W // L)
    def _group(g):
        row0 = g * L
        for r in range(L):
            w = (row0 + r) * DIM
            p0 = a_v[pl.ds(w, L)] * b_v[pl.ds(w, L)]
            p1 = a_v[pl.ds(w + L, L)] * b_v[pl.ds(w + L, L)]
            p2 = a_v[pl.ds(w + 2 * L, L)] * b_v[pl.ds(w + 2 * L, L)]
            p3 = a_v[pl.ds(w + 3 * L, L)] * b_v[pl.ds(w + 3 * L, L)]
            s_v[pl.ds(r * L, L)] = (p0 + p1) + (p2 + p3)
        res = plsc.load_gather(s_v, [col_idx[0]])
        for k in range(1, L):
            res = res + plsc.load_gather(s_v, [col_idx[k]])
        o_v[pl.ds(row0, L)] = res

    pltpu.sync_copy(o_v, o_hbm.at[pl.ds(base, B_PER_W)])


@jax.jit
def kernel(src_idx, dst_idx, in_emb, out_emb):
    mesh = plsc.VectorSubcoreMesh(
        core_axis_name="c", subcore_axis_name="s",
        num_cores=NC, num_subcores=NS)
    cp = pltpu.CompilerParams(needs_layout_passes=False)

    k1 = pl.kernel(
        _gather_body,
        out_type=(
            jax.ShapeDtypeStruct((BATCH * DIM,), jnp.float32),
            jax.ShapeDtypeStruct((BATCH * DIM,), jnp.float32),
        ),
        mesh=mesh,
        compiler_params=cp,
        scratch_types=[
            pltpu.VMEM((BATCH,), jnp.int32),
            pltpu.VMEM((BATCH,), jnp.int32),
            pltpu.VMEM((HIT_CAP,), jnp.int32),
            pltpu.VMEM((HIT_CAP,), jnp.int32),
            pltpu.VMEM((GCAP,), jnp.int32),
            pltpu.VMEM((GCAP,), jnp.int32),
            pltpu.VMEM((GCAP,), jnp.int32),
            pltpu.VMEM((GCAP,), jnp.int32),
            pltpu.VMEM((L,), jnp.int32),
            pltpu.VMEM((DIM, 128), jnp.float32),
            pltpu.VMEM((DIM, 128), jnp.float32),
            pltpu.VMEM((DIM, 128), jnp.float32),
            pltpu.VMEM((DIM, 128), jnp.float32),
            pltpu.VMEM((DIM, 128), jnp.float32),
            pltpu.VMEM((DIM, 128), jnp.float32),
            pltpu.VMEM((DIM, 128), jnp.float32),
            pltpu.VMEM((DIM, 128), jnp.float32),
            pltpu.VMEM((8, DIM), jnp.float32),
            pltpu.SemaphoreType.DMA,
            pltpu.SemaphoreType.DMA,
        ],
    )
    k2 = pl.kernel(
        _dot_body,
        out_type=jax.ShapeDtypeStruct((BATCH,), jnp.float32),
        mesh=mesh,
        compiler_params=cp,
        scratch_types=[
            pltpu.VMEM((B_PER_W * DIM,), jnp.float32),
            pltpu.VMEM((B_PER_W * DIM,), jnp.float32),
            pltpu.VMEM((L * L,), jnp.float32),
            pltpu.VMEM((B_PER_W,), jnp.float32),
        ],
    )
    gs, gd = k1(src_idx, dst_idx, in_emb.T, out_emb.T)
    return k2(gs, gd)
